# Initial kernel scaffold; baseline (speedup 1.0000x reference)
import jax
import jax.numpy as jnp
from jax.experimental import pallas as pl


def kernel(x, edge_index, batch, W1, b1, g1, bt1, W2, b2, g2, bt2, W3, b3, g3, bt3, W4, b4, g4, bt4, fW1, fb1, g5, bt5, fW2, fb2, g6, bt6, fW3, fb3):
    raise NotImplementedError("write your pallas kernel here")


# speedup vs baseline: 8.3911x; 8.3911x over previous
import functools

import jax
import jax.numpy as jnp
from jax import lax
from jax.experimental import pallas as pl
from jax.experimental.pallas import tpu as pltpu
from jax.experimental.pallas import tpu_sc as plsc

N = 10000
E = 160000
G = 64
NP = 10240
NC, NS = 2, 16
NW = NC * NS
EPW = 5120
CH = 128
NCHUNK = EPW // CH
RPT = NP // NS

_mesh = plsc.VectorSubcoreMesh(core_axis_name="c", subcore_axis_name="s",
                               num_cores=NC, num_subcores=NS)


def _sc_aggnrm(F):
    @functools.partial(
        pl.kernel,
        out_type=jax.ShapeDtypeStruct((NC * NP, F), jnp.float32),
        mesh=_mesh,
        compiler_params=pltpu.CompilerParams(use_tc_tiling_on_sc=False),
        scratch_types=[
            pltpu.VMEM((CH,), jnp.int32),
            pltpu.VMEM((CH,), jnp.int32),
            pltpu.VMEM((CH, F), jnp.float32),
            pltpu.VMEM((CH, 16), jnp.float32),
            pltpu.VMEM((CH, 16), jnp.float32),
            pltpu.VMEM((16, F), jnp.float32),
            pltpu.VMEM_SHARED((NP, F), jnp.float32),
            pltpu.SemaphoreType.DMA,
        ],
    )
    def k(xw_hbm, dv_hbm, src_hbm, dst_hbm, out_hbm,
          sidx, didx, rows, dvs, dvd, zbuf, acc, sem):
        c = lax.axis_index("c")
        s = lax.axis_index("s")
        w = c * NS + s
        zv = jnp.zeros((16,), jnp.float32)
        for i in range(16):
            for j in range(F // 16):
                zbuf[i, pl.ds(j * 16, 16)] = zv
        rbase = s * RPT
        for kk in range(RPT // 16):
            pltpu.sync_copy(zbuf, acc.at[pl.ds(rbase + kk * 16, 16)])
        plsc.subcore_barrier()

        ebase = w * EPW

        def body(i, carry):
            off = pl.multiple_of(ebase + i * CH, 8)
            pltpu.sync_copy(src_hbm.at[pl.ds(off, CH)], sidx)
            pltpu.sync_copy(dst_hbm.at[pl.ds(off, CH)], didx)
            pltpu.async_copy(xw_hbm.at[sidx], rows, sem).wait()
            pltpu.async_copy(dv_hbm.at[sidx], dvs, sem).wait()
            pltpu.async_copy(dv_hbm.at[didx], dvd, sem).wait()
            for r in range(CH):
                nv = dvs[r] * dvd[r]
                for j in range(F // 16):
                    rows[r, pl.ds(j * 16, 16)] = rows[r, pl.ds(j * 16, 16)] * nv
            pltpu.sync_copy(rows, acc.at[didx], add=True)
            return carry

        lax.fori_loop(0, NCHUNK, body, 0)
        plsc.subcore_barrier()
        obase = pl.multiple_of(c * NP + rbase, 8)
        pltpu.sync_copy(acc.at[pl.ds(rbase, RPT)], out_hbm.at[pl.ds(obase, RPT)])

    return k


def _sc_degree():
    @functools.partial(
        pl.kernel,
        out_type=jax.ShapeDtypeStruct((NC * NP, 16), jnp.float32),
        mesh=_mesh,
        compiler_params=pltpu.CompilerParams(use_tc_tiling_on_sc=False),
        scratch_types=[
            pltpu.VMEM((CH,), jnp.int32),
            pltpu.VMEM((CH, 16), jnp.float32),
            pltpu.VMEM((16, 16), jnp.float32),
            pltpu.VMEM_SHARED((NP, 16), jnp.float32),
        ],
    )
    def k(dst_hbm, out_hbm, didx, ones, zbuf, acc):
        c = lax.axis_index("c")
        s = lax.axis_index("s")
        w = c * NS + s
        ov = jnp.ones((16,), jnp.float32)
        zv = jnp.zeros((16,), jnp.float32)
        for i in range(CH):
            ones[i] = ov
        for i in range(16):
            zbuf[i] = zv
        rbase = s * RPT
        for kk in range(RPT // 16):
            pltpu.sync_copy(zbuf, acc.at[pl.ds(rbase + kk * 16, 16)])
        plsc.subcore_barrier()

        ebase = w * EPW

        def body(i, carry):
            off = pl.multiple_of(ebase + i * CH, 8)
            pltpu.sync_copy(dst_hbm.at[pl.ds(off, CH)], didx)
            pltpu.sync_copy(ones, acc.at[didx], add=True)
            return carry

        lax.fori_loop(0, NCHUNK, body, 0)
        plsc.subcore_barrier()
        obase = pl.multiple_of(c * NP + rbase, 8)
        pltpu.sync_copy(acc.at[pl.ds(rbase, RPT)], out_hbm.at[pl.ds(obase, RPT)])

    return k


def _bdot(a, b):
    return jnp.dot(a.astype(jnp.bfloat16), b.astype(jnp.bfloat16),
                   preferred_element_type=jnp.float32)


def _bn_relu(y, g, bt):
    m = jnp.mean(y, axis=0, keepdims=True)
    v = jnp.mean((y - m) * (y - m), axis=0, keepdims=True)
    return jnp.maximum((y - m) * lax.rsqrt(v + 1e-5) * g + bt, 0.0)


def _k_dinv16(deg_ref, dv_ref):
    d = deg_ref[...]
    deg = d[0:N, :] + d[NP:NP + N, :] + 1.0
    dv_ref[pl.ds(0, N), :] = lax.rsqrt(deg)
    dv_ref[pl.ds(N, NP - N), :] = jnp.zeros((NP - N, 16), jnp.float32)


def _k_lay1(x_ref, w_ref, xwp_ref):
    xw = _bdot(x_ref[...], w_ref[...])
    xwp_ref[pl.ds(0, N), :] = xw
    xwp_ref[pl.ds(N, NP - N), :] = jnp.zeros((NP - N, xw.shape[1]), jnp.float32)


def _k_mid(agg_ref, xwp_ref, dv_ref, b_ref, g_ref, bt_ref, wn_ref, out_ref):
    dv = dv_ref[0:N, 0:1]
    dv2 = dv * dv
    a = agg_ref[...]
    asum = a[0:N, :] + a[NP:NP + N, :]
    y = asum + dv2 * xwp_ref[0:N, :] + b_ref[...]
    h = _bn_relu(y, g_ref[...], bt_ref[...])
    xwn = _bdot(h, wn_ref[...])
    out_ref[pl.ds(0, N), :] = xwn
    out_ref[pl.ds(N, NP - N), :] = jnp.zeros((NP - N, xwn.shape[1]), jnp.float32)


def _k_post3(agg_ref, xwp_ref, dv_ref, b_ref, g_ref, bt_ref, out_ref):
    dv = dv_ref[0:N, 0:1]
    dv2 = dv * dv
    a = agg_ref[...]
    asum = a[0:N, :] + a[NP:NP + N, :]
    y = asum + dv2 * xwp_ref[0:N, :] + b_ref[...]
    h = _bn_relu(y, g_ref[...], bt_ref[...])
    hb = h.astype(jnp.bfloat16).astype(jnp.float32)
    out_ref[pl.ds(0, N), :] = hb
    out_ref[pl.ds(N, NP - N), :] = jnp.zeros((NP - N, 128), jnp.float32)


def _k_l4stats(agg_ref, hbp_ref, dv_ref, w_ref, b_ref, g_ref, bt_ref,
               s4_ref, ab_ref):
    dv = dv_ref[0:N, 0:1]
    dv2 = dv * dv
    a = agg_ref[...]
    asum = a[0:N, :] + a[NP:NP + N, :]
    s4 = asum + dv2 * hbp_ref[0:N, :]
    s4_ref[...] = s4
    wb = w_ref[...].astype(jnp.bfloat16).astype(jnp.float32)
    hi = lax.Precision.HIGHEST
    mbar = jnp.mean(s4, axis=0, keepdims=True)
    sc = s4 - mbar
    cc = lax.dot_general(sc, sc, (((0,), (0,)), ((), ())),
                         precision=hi) * (1.0 / N)
    amean = jnp.dot(mbar, wb, preferred_element_type=jnp.float32,
                    precision=hi)
    t = jnp.dot(cc, wb, preferred_element_type=jnp.float32, precision=hi)
    var = jnp.sum(wb * t, axis=0, keepdims=True)
    alpha = g_ref[...] * lax.rsqrt(var + 1e-5)
    beta = bt_ref[...] - (amean + b_ref[...]) * alpha
    ab_ref[0:1, :] = alpha
    ab_ref[1:2, :] = beta


RB = 400
NB = N // RB


def _k_l4pool(s4_ref, bids_ref, w_ref, b_ref, ab_ref,
              fw1_ref, fb1_ref, g5_ref, bt5_ref,
              fw2_ref, fb2_ref, g6_ref, bt6_ref,
              fw3_ref, fb3_ref, out_ref, pacc):
    i = pl.program_id(0)
    neg = jnp.float32(-jnp.inf)

    @pl.when(i == 0)
    def _():
        pacc[...] = jnp.full((G, 1024), neg, jnp.float32)

    wb = w_ref[...].astype(jnp.bfloat16).astype(jnp.float32)
    y = jnp.dot(s4_ref[...], wb, preferred_element_type=jnp.float32,
                precision=lax.Precision.HIGHEST)
    y = y + b_ref[...]
    h = jnp.maximum(y * ab_ref[0:1, :] + ab_ref[1:2, :], 0.0)
    bids = bids_ref[0, :, :]
    glo = bids_ref[0, 0, 0]
    ghi = bids_ref[0, RB - 1, 0]
    for g in range(G):
        @pl.when((glo <= g) & (g <= ghi))
        def _():
            mask = bids == g
            colmax = jnp.max(jnp.where(mask, h, neg), axis=0, keepdims=True)
            pacc[g:g + 1, :] = jnp.maximum(pacc[g:g + 1, :], colmax)

    @pl.when(i == NB - 1)
    def _():
        p = pacc[...]

        def bn(y2, gg, bb):
            m = jnp.mean(y2, axis=0, keepdims=True)
            v = jnp.mean((y2 - m) * (y2 - m), axis=0, keepdims=True)
            return (y2 - m) * lax.rsqrt(v + 1e-5) * gg + bb

        hh = _bdot(p, fw1_ref[...])
        hh = jnp.maximum(bn(hh + fb1_ref[...], g5_ref[...], bt5_ref[...]), 0.0)
        hh = _bdot(hh, fw2_ref[...])
        hh = jnp.maximum(bn(hh + fb2_ref[...], g6_ref[...], bt6_ref[...]), 0.0)
        hh = _bdot(hh, fw3_ref[...])
        hh = hh + fb3_ref[...]
        nn = jnp.sqrt(jnp.sum(hh * hh, axis=1, keepdims=True))
        out_ref[...] = hh / jnp.maximum(nn, 1e-12)


def kernel(x, edge_index, batch, W1, b1, g1, bt1, W2, b2, g2, bt2,
           W3, b3, g3, bt3, W4, b4, g4, bt4, fW1, fb1, g5, bt5,
           fW2, fb2, g6, bt6, fW3, fb3):
    f32 = jnp.float32
    epw_real = E // NW
    src = edge_index[0].reshape(NW, epw_real)
    dst = edge_index[1].reshape(NW, epw_real)
    padrow = (N + jnp.arange(NW, dtype=jnp.int32))[:, None]
    pad = jnp.broadcast_to(padrow, (NW, EPW - epw_real))
    src_p = jnp.concatenate([src, pad], axis=1).reshape(-1)
    dst_p = jnp.concatenate([dst, pad], axis=1).reshape(-1)
    bids3d = batch.reshape(NB, RB, 1)
    r = lambda v: v.reshape(1, -1)

    deg = _sc_degree()(dst_p)
    dv16 = pl.pallas_call(
        _k_dinv16, out_shape=jax.ShapeDtypeStruct((NP, 16), f32))(deg)

    xwp1 = pl.pallas_call(
        _k_lay1, out_shape=jax.ShapeDtypeStruct((NP, 32), f32))(x, W1)
    agg1 = _sc_aggnrm(32)(xwp1, dv16, src_p, dst_p)

    xwp2 = pl.pallas_call(
        _k_mid, out_shape=jax.ShapeDtypeStruct((NP, 64), f32),
    )(agg1, xwp1, dv16, r(b1), r(g1), r(bt1), W2)
    agg2 = _sc_aggnrm(64)(xwp2, dv16, src_p, dst_p)

    xwp3 = pl.pallas_call(
        _k_mid, out_shape=jax.ShapeDtypeStruct((NP, 128), f32),
    )(agg2, xwp2, dv16, r(b2), r(g2), r(bt2), W3)
    agg3 = _sc_aggnrm(128)(xwp3, dv16, src_p, dst_p)

    hbp3 = pl.pallas_call(
        _k_post3, out_shape=jax.ShapeDtypeStruct((NP, 128), f32),
    )(agg3, xwp3, dv16, r(b3), r(g3), r(bt3))
    agg4 = _sc_aggnrm(128)(hbp3, dv16, src_p, dst_p)

    s4, ab = pl.pallas_call(
        _k_l4stats,
        out_shape=[jax.ShapeDtypeStruct((N, 128), f32),
                   jax.ShapeDtypeStruct((2, 1024), f32)],
    )(agg4, hbp3, dv16, W4, r(b4), r(g4), r(bt4))

    const = lambda *bs: pl.BlockSpec(bs, lambda i: tuple(0 for _ in bs))
    out = pl.pallas_call(
        _k_l4pool,
        grid=(NB,),
        in_specs=[
            pl.BlockSpec((RB, 128), lambda i: (i, 0)),
            pl.BlockSpec((1, RB, 1), lambda i: (i, 0, 0)),
            const(128, 1024), const(1, 1024), const(2, 1024),
            const(1024, 512), const(1, 512), const(1, 512), const(1, 512),
            const(512, 256), const(1, 256), const(1, 256), const(1, 256),
            const(256, 64), const(1, 64),
        ],
        out_specs=pl.BlockSpec((G, 64), lambda i: (0, 0)),
        out_shape=jax.ShapeDtypeStruct((G, 64), f32),
        scratch_shapes=[pltpu.VMEM((G, 1024), f32)],
    )(s4, bids3d, W4, r(b4), ab,
      fW1, r(fb1), r(g5), r(bt5), fW2, r(fb2), r(g6), r(bt6), fW3, r(fb3))
    return out
